# baseline (speedup 1.0000x reference)
import functools

import jax
import jax.numpy as jnp
from jax import lax
from jax.experimental import pallas as pl
from jax.experimental.pallas import tpu as pltpu
from jax.experimental.pallas import tpu_sc as plsc

NC = 2
NS = 16
NW = NC * NS
GB = 128

_STEPS = 4


def _mesh():
    return plsc.VectorSubcoreMesh(
        core_axis_name="c", subcore_axis_name="s", num_cores=NC, num_subcores=NS
    )


@functools.lru_cache(maxsize=None)
def _make_gather(N, E, D):
    assert E % NW == 0
    epw = E // NW
    GBG = 64
    nb = epw // GBG
    tail = epw - nb * GBG
    R = 4
    DIST = 2
    assert epw % 8 == 0 and tail % 8 == 0 and GBG % 8 == 0
    T = nb // R
    nb8 = T * R
    rpt = -(-(N // NS) // 8) * 8
    rlast = N - (NS - 1) * rpt
    assert 0 < rlast <= rpt

    def body(h_hbm, src_hbm, out_hbm, idx_all, r0, r1, r2, r3,
             idx_t, rows_t, htab, g0, g1, g2, g3,
             w0, w1, w2, w3, sem):
        c = lax.axis_index("c")
        s = lax.axis_index("s")
        base = pl.multiple_of(((s * NC + c) * epw).astype(jnp.int32), 8)
        rows = (r0, r1, r2, r3)
        gsems = (g0, g1, g2, g3)
        wsems = (w0, w1, w2, w3)
        hr0 = pl.multiple_of(s * rpt, 8)

        @pl.when(s < NS - 1)
        def _():
            pltpu.sync_copy(h_hbm.at[pl.ds(hr0, rpt)], htab.at[pl.ds(hr0, rpt)])

        @pl.when(s == NS - 1)
        def _():
            pltpu.sync_copy(
                h_hbm.at[pl.ds(hr0, rlast)], htab.at[pl.ds(hr0, rlast)]
            )

        pltpu.sync_copy(src_hbm.at[pl.ds(base, epw)], idx_all)
        plsc.subcore_barrier()

        def fire_gather(j, b):
            pltpu.async_copy(
                htab.at[idx_all.at[pl.ds(pl.multiple_of(j * GBG, 8), GBG)]],
                rows[b], gsems[b],
            )

        def wait_gather(b):
            pltpu.make_async_copy(htab.at[pl.ds(0, GBG)], rows[b], gsems[b]).wait()

        def fire_wb(j, b):
            off = pl.multiple_of(base + j * GBG, 8)
            pltpu.async_copy(rows[b], out_hbm.at[pl.ds(off, GBG)], wsems[b])

        def wait_wb(b):
            pltpu.make_async_copy(rows[b], out_hbm.at[pl.ds(base, GBG)], wsems[b]).wait()

        if T:
            for b in range(DIST):
                fire_gather(b, b)

            def outer(t, carry):
                j0 = t * R
                for b in range(R):
                    j = j0 + b
                    jw = j - DIST
                    bw = (b - DIST) % R

                    @pl.when(j - R >= 0)
                    def _():
                        wait_wb(b)

                    @pl.when(j >= DIST)
                    def _():
                        fire_gather(j, b)

                    @pl.when(jw >= 0)
                    def _():
                        wait_gather(bw)
                        fire_wb(jw, bw)
                return carry

            lax.fori_loop(0, T, outer, 0)
            for j in range(nb8 - DIST, nb8):
                b = j % R
                wait_gather(b)
                fire_wb(j, b)
            for j in range(max(nb8 - R, 0), nb8):
                wait_wb(j % R)
        for j in range(nb8, nb):
            off = pl.multiple_of(base + j * GBG, 8)
            pltpu.async_copy(
                htab.at[idx_all.at[pl.ds(j * GBG, GBG)]], r0, sem
            ).wait()
            pltpu.sync_copy(r0, out_hbm.at[pl.ds(off, GBG)])
        if tail:
            off = pl.multiple_of(base + nb * GBG, 8)
            pltpu.sync_copy(src_hbm.at[pl.ds(off, tail)], idx_t)
            pltpu.async_copy(htab.at[idx_t], rows_t, sem).wait()
            pltpu.sync_copy(rows_t, out_hbm.at[pl.ds(off, tail)])

    return pl.kernel(
        body,
        out_type=jax.ShapeDtypeStruct((E, D), jnp.float32),
        mesh=_mesh(),
        scratch_types=(
            [pltpu.VMEM((epw,), jnp.int32)]
            + [pltpu.VMEM((GBG, D), jnp.float32) for _ in range(4)]
            + [
                pltpu.VMEM((max(tail, 8),), jnp.int32),
                pltpu.VMEM((max(tail, 8), D), jnp.float32),
                pltpu.VMEM_SHARED((N, D), jnp.float32),
            ]
            + [pltpu.SemaphoreType.DMA for _ in range(9)]
        ),
    )


@functools.lru_cache(maxsize=None)
def _make_segsum(N, E, D):
    epw = E // NW
    nb = epw // GB
    tail = epw - nb * GB
    rpt = -(-(N // NS) // 8) * 8
    rlast = N - (NS - 1) * rpt
    assert E % NW == 0 and tail % 8 == 0 and 0 < rlast <= rpt

    R = 2
    T = nb // R
    rem = nb - T * R
    ZR = 56

    def body(t_hbm, dst_hbm, agg_hbm, b0, b1, i0, i1, buf_t, idx_t, zbuf, acc,
             rs0, rs1, ss0, ss1, sem):
        c = lax.axis_index("c")
        s = lax.axis_index("s")
        bufs = (b0, b1)
        ibufs = (i0, i1)
        rsems = (rs0, rs1)
        ssems = (ss0, ss1)

        def zstep(i, carry):
            r = i // (D // 16)
            q = (i % (D // 16)) * 16
            zbuf[r, pl.ds(q, 16)] = jnp.zeros((16,), jnp.float32)
            return carry

        lax.fori_loop(0, ZR * (D // 16), zstep, 0)
        r0 = pl.multiple_of(s * rpt, 8)
        nrep = lax.select(s < NS - 1, rpt // ZR, rlast // ZR)

        def zcopy(r, carry):
            pltpu.sync_copy(zbuf, acc.at[pl.ds(r0 + r * ZR, ZR)])
            return carry

        lax.fori_loop(0, nrep, zcopy, 0)
        zr_done = nrep * ZR
        zrem1 = rpt - (rpt // ZR) * ZR
        zrem2 = rlast - (rlast // ZR) * ZR
        assert zrem1 == zrem2 and zrem1 > 0
        pltpu.sync_copy(zbuf.at[pl.ds(0, zrem1)], acc.at[pl.ds(r0 + zr_done, zrem1)])
        plsc.subcore_barrier()

        base = pl.multiple_of(((c * NS + s) * epw).astype(jnp.int32), 8)

        def fire_reads(j, b):
            off = pl.multiple_of(base + j * GB, 8)
            pltpu.async_copy(t_hbm.at[pl.ds(off, GB)], bufs[b], rsems[b])
            pltpu.async_copy(dst_hbm.at[pl.ds(off, GB)], ibufs[b], rsems[b])

        for b in range(R):
            fire_reads(b, b)

        def outer(t, carry):
            for b in range(R):
                pltpu.make_async_copy(
                    t_hbm.at[pl.ds(0, GB)], bufs[b], rsems[b]
                ).wait()
                pltpu.make_async_copy(
                    dst_hbm.at[pl.ds(0, GB)], ibufs[b], rsems[b]
                ).wait()
                pltpu.async_copy(bufs[b], acc.at[ibufs[b]], ssems[b], add=True)
            for b in range(R):
                @pl.when(t < T - 1)
                def _():
                    pltpu.make_async_copy(
                        t_hbm.at[pl.ds(0, GB)], bufs[b], ssems[b]
                    ).wait()
                    fire_reads((t + 1) * R + b, b)
            return carry

        lax.fori_loop(0, T, outer, 0)
        for b in range(R):
            pltpu.make_async_copy(t_hbm.at[pl.ds(0, GB)], bufs[b], ssems[b]).wait()
        for j in range(T * R, nb):
            off = pl.multiple_of(base + j * GB, 8)
            pltpu.sync_copy(t_hbm.at[pl.ds(off, GB)], b0)
            pltpu.sync_copy(dst_hbm.at[pl.ds(off, GB)], i0)
            pltpu.sync_copy(b0, acc.at[i0], add=True)
        if tail:
            off = pl.multiple_of(base + nb * GB, 8)
            pltpu.sync_copy(t_hbm.at[pl.ds(off, tail)], buf_t)
            pltpu.sync_copy(dst_hbm.at[pl.ds(off, tail)], idx_t)
            pltpu.sync_copy(buf_t, acc.at[idx_t], add=True)
        plsc.subcore_barrier()

        @pl.when(s < NS - 1)
        def _():
            pltpu.sync_copy(
                acc.at[pl.ds(r0, rpt)], agg_hbm.at[c, pl.ds(r0, rpt)]
            )

        @pl.when(s == NS - 1)
        def _():
            pltpu.sync_copy(
                acc.at[pl.ds(r0, rlast)], agg_hbm.at[c, pl.ds(r0, rlast)]
            )

    return pl.kernel(
        body,
        out_type=jax.ShapeDtypeStruct((NC, N, D), jnp.float32),
        mesh=_mesh(),
        scratch_types=[
            pltpu.VMEM((GB, D), jnp.float32),
            pltpu.VMEM((GB, D), jnp.float32),
            pltpu.VMEM((GB,), jnp.int32),
            pltpu.VMEM((GB,), jnp.int32),
            pltpu.VMEM((max(tail, 8), D), jnp.float32),
            pltpu.VMEM((max(tail, 8),), jnp.int32),
            pltpu.VMEM((ZR, D), jnp.float32),
            pltpu.VMEM_SHARED((N, D), jnp.float32),
            pltpu.SemaphoreType.DMA,
            pltpu.SemaphoreType.DMA,
            pltpu.SemaphoreType.DMA,
            pltpu.SemaphoreType.DMA,
            pltpu.SemaphoreType.DMA,
        ],
    )


def _edge_transform(xj, bond, wstack, off_rows=0):
    D = xj.shape[1]
    rows = bond.shape[0]
    BDp1 = wstack.shape[0]
    be = 2048
    assert off_rows % be == 0
    ob = off_rows // be
    grid = -(-rows // be)

    def body(xj_ref, b_ref, w_ref, o_ref):
        x = xj_ref[...]
        acc = jnp.dot(x, w_ref[0], preferred_element_type=jnp.float32)
        for k in range(1, BDp1):
            acc += b_ref[:, k - 1 : k] * jnp.dot(
                x, w_ref[k], preferred_element_type=jnp.float32
            )
        o_ref[...] = acc

    return pl.pallas_call(
        body,
        grid=(grid,),
        in_specs=[
            pl.BlockSpec((be, D), lambda i: (i + ob, 0)),
            pl.BlockSpec((be, bond.shape[1]), lambda i: (i, 0)),
            pl.BlockSpec((BDp1, D, D), lambda i: (0, 0, 0)),
        ],
        out_specs=pl.BlockSpec((be, D), lambda i: (i, 0)),
        out_shape=jax.ShapeDtypeStruct((rows, D), jnp.float32),
    )(xj, bond, wstack)


def _gru(aggs, h, wk, wr, b):
    N, D = h.shape
    bn = 2000
    assert N % bn == 0
    na = len(aggs)

    def body(*refs):
        a_refs = refs[:na]
        h_ref, wk_ref, wr_ref, b_ref, o_ref = refs[na:]
        a = a_refs[0][0] + a_refs[0][1]
        for ar in a_refs[1:]:
            a = a + ar[0] + ar[1]
        hh = h_ref[...]
        xp = jnp.dot(a, wk_ref[...], preferred_element_type=jnp.float32) + b_ref[0]
        hp = jnp.dot(hh, wr_ref[...], preferred_element_type=jnp.float32) + b_ref[1]
        z = jax.nn.sigmoid(xp[:, :D] + hp[:, :D])
        r = jax.nn.sigmoid(xp[:, D : 2 * D] + hp[:, D : 2 * D])
        cand = jnp.tanh(xp[:, 2 * D :] + r * hp[:, 2 * D :])
        o_ref[...] = z * hh + (1.0 - z) * cand

    return pl.pallas_call(
        body,
        grid=(N // bn,),
        in_specs=[pl.BlockSpec((NC, bn, D), lambda i: (0, i, 0)) for _ in range(na)]
        + [
            pl.BlockSpec((bn, D), lambda i: (i, 0)),
            pl.BlockSpec((D, 3 * D), lambda i: (0, 0)),
            pl.BlockSpec((D, 3 * D), lambda i: (0, 0)),
            pl.BlockSpec((2, 3 * D), lambda i: (0, 0)),
        ],
        out_specs=pl.BlockSpec((bn, D), lambda i: (i, 0)),
        out_shape=jax.ShapeDtypeStruct((N, D), jnp.float32),
    )(*aggs, h, wk, wr, b)


def kernel(atom_features, bond_features, pair_indices, kernel, bias, gru_kernel,
           gru_rec_kernel, gru_bias):
    N, D = atom_features.shape
    E, BD = bond_features.shape
    dst = pair_indices[:, 0].astype(jnp.int32)
    src = pair_indices[:, 1].astype(jnp.int32)
    Kr = kernel.reshape(BD, D, D)
    wstack = jnp.concatenate(
        [bias.reshape(D, D).T[None], jnp.transpose(Kr, (0, 2, 1))], axis=0
    )
    EA = (E // (2 * 2048)) * 2048
    assert (EA // NW) % 8 == 0 and ((E - EA) // NW) % 8 == 0
    if 0 < EA < E:
        bounds = [(0, EA), (EA, E)]
    else:
        bounds = [(0, E)]
    parts = [
        (lo, dst[lo:hi], bond_features[lo:hi], _make_segsum(N, hi - lo, D))
        for lo, hi in bounds
    ]
    gather = _make_gather(N, E, D)

    h = atom_features
    for _ in range(_STEPS):
        xj = gather(h, src)
        aggs = []
        for lo, d_, bf, seg in parts:
            t = _edge_transform(xj, bf, wstack, off_rows=lo)
            aggs.append(seg(t, d_))
        h = _gru(aggs, h, gru_kernel, gru_rec_kernel, gru_bias)
    return h


# speedup vs baseline: 1.0377x; 1.0377x over previous
import functools

import jax
import jax.numpy as jnp
from jax import lax
from jax.experimental import pallas as pl
from jax.experimental.pallas import tpu as pltpu
from jax.experimental.pallas import tpu_sc as plsc

NC = 2
NS = 16
NW = NC * NS
GB = 128

_STEPS = 4


def _mesh():
    return plsc.VectorSubcoreMesh(
        core_axis_name="c", subcore_axis_name="s", num_cores=NC, num_subcores=NS
    )


@functools.lru_cache(maxsize=None)
def _make_gather(N, E, D):
    assert E % NW == 0
    epw = E // NW
    GBG = 64
    nb = epw // GBG
    tail = epw - nb * GBG
    R = 4
    DIST = 2
    assert epw % 8 == 0 and tail % 8 == 0 and GBG % 8 == 0
    T = nb // R
    nb8 = T * R
    rpt = -(-(N // NS) // 8) * 8
    rlast = N - (NS - 1) * rpt
    assert 0 < rlast <= rpt

    def body(h_hbm, src_hbm, out_hbm, idx_all, r0, r1, r2, r3,
             idx_t, rows_t, htab, g0, g1, g2, g3,
             w0, w1, w2, w3, sem):
        c = lax.axis_index("c")
        s = lax.axis_index("s")
        base = pl.multiple_of(((s * NC + c) * epw).astype(jnp.int32), 8)
        rows = (r0, r1, r2, r3)
        gsems = (g0, g1, g2, g3)
        wsems = (w0, w1, w2, w3)
        hr0 = pl.multiple_of(s * rpt, 8)

        @pl.when(s < NS - 1)
        def _():
            pltpu.sync_copy(h_hbm.at[pl.ds(hr0, rpt)], htab.at[pl.ds(hr0, rpt)])

        @pl.when(s == NS - 1)
        def _():
            pltpu.sync_copy(
                h_hbm.at[pl.ds(hr0, rlast)], htab.at[pl.ds(hr0, rlast)]
            )

        pltpu.sync_copy(src_hbm.at[pl.ds(base, epw)], idx_all)
        plsc.subcore_barrier()

        def fire_gather(j, b):
            pltpu.async_copy(
                htab.at[idx_all.at[pl.ds(pl.multiple_of(j * GBG, 8), GBG)]],
                rows[b], gsems[b],
            )

        def wait_gather(b):
            pltpu.make_async_copy(htab.at[pl.ds(0, GBG)], rows[b], gsems[b]).wait()

        def fire_wb(j, b):
            off = pl.multiple_of(base + j * GBG, 8)
            pltpu.async_copy(rows[b], out_hbm.at[pl.ds(off, GBG)], wsems[b])

        def wait_wb(b):
            pltpu.make_async_copy(rows[b], out_hbm.at[pl.ds(base, GBG)], wsems[b]).wait()

        if T:
            for b in range(DIST):
                fire_gather(b, b)

            def outer(t, carry):
                j0 = t * R
                for b in range(R):
                    j = j0 + b
                    jw = j - DIST
                    bw = (b - DIST) % R

                    @pl.when(j - R >= 0)
                    def _():
                        wait_wb(b)

                    @pl.when(j >= DIST)
                    def _():
                        fire_gather(j, b)

                    @pl.when(jw >= 0)
                    def _():
                        wait_gather(bw)
                        fire_wb(jw, bw)
                return carry

            lax.fori_loop(0, T, outer, 0)
            for j in range(nb8 - DIST, nb8):
                b = j % R
                wait_gather(b)
                fire_wb(j, b)
            for j in range(max(nb8 - R, 0), nb8):
                wait_wb(j % R)
        for j in range(nb8, nb):
            off = pl.multiple_of(base + j * GBG, 8)
            pltpu.async_copy(
                htab.at[idx_all.at[pl.ds(j * GBG, GBG)]], r0, sem
            ).wait()
            pltpu.sync_copy(r0, out_hbm.at[pl.ds(off, GBG)])
        if tail:
            off = pl.multiple_of(base + nb * GBG, 8)
            pltpu.sync_copy(src_hbm.at[pl.ds(off, tail)], idx_t)
            pltpu.async_copy(htab.at[idx_t], rows_t, sem).wait()
            pltpu.sync_copy(rows_t, out_hbm.at[pl.ds(off, tail)])

    return pl.kernel(
        body,
        out_type=jax.ShapeDtypeStruct((E, D), jnp.float32),
        mesh=_mesh(),
        scratch_types=(
            [pltpu.VMEM((epw,), jnp.int32)]
            + [pltpu.VMEM((GBG, D), jnp.float32) for _ in range(4)]
            + [
                pltpu.VMEM((max(tail, 8),), jnp.int32),
                pltpu.VMEM((max(tail, 8), D), jnp.float32),
                pltpu.VMEM_SHARED((N, D), jnp.float32),
            ]
            + [pltpu.SemaphoreType.DMA for _ in range(9)]
        ),
    )


@functools.lru_cache(maxsize=None)
def _make_segsum(N, E, D):
    epw = E // NW
    nb = epw // GB
    tail = epw - nb * GB
    rpt = -(-(N // NS) // 8) * 8
    rlast = N - (NS - 1) * rpt
    assert E % NW == 0 and tail % 8 == 0 and 0 < rlast <= rpt

    R = 2
    T = nb // R
    rem = nb - T * R
    ZR = 56

    def body(t_hbm, dst_hbm, agg_hbm, b0, b1, i0, i1, buf_t, idx_t, zbuf, acc,
             rs0, rs1, ss0, ss1, sem):
        c = lax.axis_index("c")
        s = lax.axis_index("s")
        bufs = (b0, b1)
        ibufs = (i0, i1)
        rsems = (rs0, rs1)
        ssems = (ss0, ss1)

        def zstep(i, carry):
            r = i // (D // 16)
            q = (i % (D // 16)) * 16
            zbuf[r, pl.ds(q, 16)] = jnp.zeros((16,), jnp.float32)
            return carry

        lax.fori_loop(0, ZR * (D // 16), zstep, 0)
        r0 = pl.multiple_of(s * rpt, 8)
        nrep = lax.select(s < NS - 1, rpt // ZR, rlast // ZR)

        def zcopy(r, carry):
            pltpu.sync_copy(zbuf, acc.at[pl.ds(r0 + r * ZR, ZR)])
            return carry

        lax.fori_loop(0, nrep, zcopy, 0)
        zr_done = nrep * ZR
        zrem1 = rpt - (rpt // ZR) * ZR
        zrem2 = rlast - (rlast // ZR) * ZR
        assert zrem1 == zrem2 and zrem1 > 0
        pltpu.sync_copy(zbuf.at[pl.ds(0, zrem1)], acc.at[pl.ds(r0 + zr_done, zrem1)])
        plsc.subcore_barrier()

        base = pl.multiple_of(((c * NS + s) * epw).astype(jnp.int32), 8)

        def fire_reads(j, b):
            off = pl.multiple_of(base + j * GB, 8)
            pltpu.async_copy(t_hbm.at[pl.ds(off, GB)], bufs[b], rsems[b])
            pltpu.async_copy(dst_hbm.at[pl.ds(off, GB)], ibufs[b], rsems[b])

        for b in range(R):
            fire_reads(b, b)

        def outer(t, carry):
            for b in range(R):
                pltpu.make_async_copy(
                    t_hbm.at[pl.ds(0, GB)], bufs[b], rsems[b]
                ).wait()
                pltpu.make_async_copy(
                    dst_hbm.at[pl.ds(0, GB)], ibufs[b], rsems[b]
                ).wait()
                pltpu.async_copy(bufs[b], acc.at[ibufs[b]], ssems[b], add=True)
            for b in range(R):
                @pl.when(t < T - 1)
                def _():
                    pltpu.make_async_copy(
                        t_hbm.at[pl.ds(0, GB)], bufs[b], ssems[b]
                    ).wait()
                    fire_reads((t + 1) * R + b, b)
            return carry

        lax.fori_loop(0, T, outer, 0)
        for b in range(R):
            pltpu.make_async_copy(t_hbm.at[pl.ds(0, GB)], bufs[b], ssems[b]).wait()
        for j in range(T * R, nb):
            off = pl.multiple_of(base + j * GB, 8)
            pltpu.sync_copy(t_hbm.at[pl.ds(off, GB)], b0)
            pltpu.sync_copy(dst_hbm.at[pl.ds(off, GB)], i0)
            pltpu.sync_copy(b0, acc.at[i0], add=True)
        if tail:
            off = pl.multiple_of(base + nb * GB, 8)
            pltpu.sync_copy(t_hbm.at[pl.ds(off, tail)], buf_t)
            pltpu.sync_copy(dst_hbm.at[pl.ds(off, tail)], idx_t)
            pltpu.sync_copy(buf_t, acc.at[idx_t], add=True)
        plsc.subcore_barrier()

        @pl.when(s < NS - 1)
        def _():
            pltpu.sync_copy(
                acc.at[pl.ds(r0, rpt)], agg_hbm.at[c, pl.ds(r0, rpt)]
            )

        @pl.when(s == NS - 1)
        def _():
            pltpu.sync_copy(
                acc.at[pl.ds(r0, rlast)], agg_hbm.at[c, pl.ds(r0, rlast)]
            )

    return pl.kernel(
        body,
        out_type=jax.ShapeDtypeStruct((NC, N, D), jnp.float32),
        mesh=_mesh(),
        scratch_types=[
            pltpu.VMEM((GB, D), jnp.float32),
            pltpu.VMEM((GB, D), jnp.float32),
            pltpu.VMEM((GB,), jnp.int32),
            pltpu.VMEM((GB,), jnp.int32),
            pltpu.VMEM((max(tail, 8), D), jnp.float32),
            pltpu.VMEM((max(tail, 8),), jnp.int32),
            pltpu.VMEM((ZR, D), jnp.float32),
            pltpu.VMEM_SHARED((N, D), jnp.float32),
            pltpu.SemaphoreType.DMA,
            pltpu.SemaphoreType.DMA,
            pltpu.SemaphoreType.DMA,
            pltpu.SemaphoreType.DMA,
            pltpu.SemaphoreType.DMA,
        ],
    )


def _edge_transform(xj, bond, wstack, off_rows=0):
    D = xj.shape[1]
    rows = bond.shape[0]
    BDp1 = wstack.shape[0]
    be = 2048
    assert off_rows % be == 0
    ob = off_rows // be
    grid = -(-rows // be)

    def body(xj_ref, b_ref, w_ref, o_ref):
        x = xj_ref[...]
        acc = jnp.dot(x, w_ref[0], preferred_element_type=jnp.float32)
        for k in range(1, BDp1):
            acc += b_ref[:, k - 1 : k] * jnp.dot(
                x, w_ref[k], preferred_element_type=jnp.float32
            )
        o_ref[...] = acc

    return pl.pallas_call(
        body,
        grid=(grid,),
        in_specs=[
            pl.BlockSpec((be, D), lambda i: (i + ob, 0)),
            pl.BlockSpec((be, bond.shape[1]), lambda i: (i, 0)),
            pl.BlockSpec((BDp1, D, D), lambda i: (0, 0, 0)),
        ],
        out_specs=pl.BlockSpec((be, D), lambda i: (i, 0)),
        out_shape=jax.ShapeDtypeStruct((rows, D), jnp.float32),
    )(xj, bond, wstack)


def _gru(aggs, h, wk, wr, b):
    N, D = h.shape
    bn = 2000
    assert N % bn == 0
    na = len(aggs)

    def body(*refs):
        a_refs = refs[:na]
        h_ref, wk_ref, wr_ref, b_ref, o_ref = refs[na:]
        a = a_refs[0][0] + a_refs[0][1]
        for ar in a_refs[1:]:
            a = a + ar[0] + ar[1]
        hh = h_ref[...]
        xp = jnp.dot(a, wk_ref[...], preferred_element_type=jnp.float32) + b_ref[0]
        hp = jnp.dot(hh, wr_ref[...], preferred_element_type=jnp.float32) + b_ref[1]
        z = jax.nn.sigmoid(xp[:, :D] + hp[:, :D])
        r = jax.nn.sigmoid(xp[:, D : 2 * D] + hp[:, D : 2 * D])
        cand = jnp.tanh(xp[:, 2 * D :] + r * hp[:, 2 * D :])
        o_ref[...] = z * hh + (1.0 - z) * cand

    return pl.pallas_call(
        body,
        grid=(N // bn,),
        in_specs=[pl.BlockSpec((NC, bn, D), lambda i: (0, i, 0)) for _ in range(na)]
        + [
            pl.BlockSpec((bn, D), lambda i: (i, 0)),
            pl.BlockSpec((D, 3 * D), lambda i: (0, 0)),
            pl.BlockSpec((D, 3 * D), lambda i: (0, 0)),
            pl.BlockSpec((2, 3 * D), lambda i: (0, 0)),
        ],
        out_specs=pl.BlockSpec((bn, D), lambda i: (i, 0)),
        out_shape=jax.ShapeDtypeStruct((N, D), jnp.float32),
    )(*aggs, h, wk, wr, b)


def kernel(atom_features, bond_features, pair_indices, kernel, bias, gru_kernel,
           gru_rec_kernel, gru_bias):
    N, D = atom_features.shape
    E, BD = bond_features.shape
    dst = pair_indices[:, 0].astype(jnp.int32)
    src = pair_indices[:, 1].astype(jnp.int32)
    Kr = kernel.reshape(BD, D, D)
    wstack = jnp.concatenate(
        [bias.reshape(D, D).T[None], jnp.transpose(Kr, (0, 2, 1))], axis=0
    )
    EA = (E // (2 * NW * GB)) * (NW * GB)
    if 0 < EA < E:
        bounds = [(0, EA), (EA, E)]
    else:
        bounds = [(0, E)]
    parts = [
        (src[lo:hi], dst[lo:hi], bond_features[lo:hi],
         _make_gather(N, hi - lo, D), _make_segsum(N, hi - lo, D))
        for lo, hi in bounds
    ]

    h = atom_features
    for _ in range(_STEPS):
        xjs = [g(h, s_) for s_, _, _, g, _ in parts]
        aggs = []
        for i, (s_, d_, bf, g, seg) in enumerate(parts):
            t = _edge_transform(xjs[i], bf, wstack)
            aggs.append(seg(t, d_))
        h = _gru(aggs, h, gru_kernel, gru_rec_kernel, gru_bias)
    return h


# speedup vs baseline: 1.0586x; 1.0202x over previous
import functools

import jax
import jax.numpy as jnp
from jax import lax
from jax.experimental import pallas as pl
from jax.experimental.pallas import tpu as pltpu
from jax.experimental.pallas import tpu_sc as plsc

NC = 2
NS = 16
NW = NC * NS
GB = 128

_STEPS = 4


def _mesh():
    return plsc.VectorSubcoreMesh(
        core_axis_name="c", subcore_axis_name="s", num_cores=NC, num_subcores=NS
    )


@functools.lru_cache(maxsize=None)
def _make_gather(N, E, D):
    assert E % NW == 0
    epw = E // NW
    GBG = 64
    nb = epw // GBG
    tail = epw - nb * GBG
    R = 4
    DIST = 2
    assert epw % 8 == 0 and tail % 8 == 0 and GBG % 8 == 0
    T = nb // R
    nb8 = T * R
    rpt = -(-(N // NS) // 8) * 8
    rlast = N - (NS - 1) * rpt
    assert 0 < rlast <= rpt

    def body(h_hbm, src_hbm, out_hbm, idx_all, r0, r1, r2, r3,
             idx_t, rows_t, htab, g0, g1, g2, g3,
             w0, w1, w2, w3, sem):
        c = lax.axis_index("c")
        s = lax.axis_index("s")
        base = pl.multiple_of(((s * NC + c) * epw).astype(jnp.int32), 8)
        rows = (r0, r1, r2, r3)
        gsems = (g0, g1, g2, g3)
        wsems = (w0, w1, w2, w3)
        hr0 = pl.multiple_of(s * rpt, 8)

        @pl.when(s < NS - 1)
        def _():
            pltpu.sync_copy(h_hbm.at[pl.ds(hr0, rpt)], htab.at[pl.ds(hr0, rpt)])

        @pl.when(s == NS - 1)
        def _():
            pltpu.sync_copy(
                h_hbm.at[pl.ds(hr0, rlast)], htab.at[pl.ds(hr0, rlast)]
            )

        pltpu.sync_copy(src_hbm.at[pl.ds(base, epw)], idx_all)
        plsc.subcore_barrier()

        def fire_gather(j, b):
            pltpu.async_copy(
                htab.at[idx_all.at[pl.ds(pl.multiple_of(j * GBG, 8), GBG)]],
                rows[b], gsems[b],
            )

        def wait_gather(b):
            pltpu.make_async_copy(htab.at[pl.ds(0, GBG)], rows[b], gsems[b]).wait()

        def fire_wb(j, b):
            off = pl.multiple_of(base + j * GBG, 8)
            pltpu.async_copy(rows[b], out_hbm.at[pl.ds(off, GBG)], wsems[b])

        def wait_wb(b):
            pltpu.make_async_copy(rows[b], out_hbm.at[pl.ds(base, GBG)], wsems[b]).wait()

        if T:
            for b in range(DIST):
                fire_gather(b, b)

            def outer(t, carry):
                j0 = t * R
                for b in range(R):
                    j = j0 + b
                    jw = j - DIST
                    bw = (b - DIST) % R

                    @pl.when(j - R >= 0)
                    def _():
                        wait_wb(b)

                    @pl.when(j >= DIST)
                    def _():
                        fire_gather(j, b)

                    @pl.when(jw >= 0)
                    def _():
                        wait_gather(bw)
                        fire_wb(jw, bw)
                return carry

            lax.fori_loop(0, T, outer, 0)
            for j in range(nb8 - DIST, nb8):
                b = j % R
                wait_gather(b)
                fire_wb(j, b)
            for j in range(max(nb8 - R, 0), nb8):
                wait_wb(j % R)
        for j in range(nb8, nb):
            off = pl.multiple_of(base + j * GBG, 8)
            pltpu.async_copy(
                htab.at[idx_all.at[pl.ds(j * GBG, GBG)]], r0, sem
            ).wait()
            pltpu.sync_copy(r0, out_hbm.at[pl.ds(off, GBG)])
        if tail:
            off = pl.multiple_of(base + nb * GBG, 8)
            pltpu.sync_copy(src_hbm.at[pl.ds(off, tail)], idx_t)
            pltpu.async_copy(htab.at[idx_t], rows_t, sem).wait()
            pltpu.sync_copy(rows_t, out_hbm.at[pl.ds(off, tail)])

    return pl.kernel(
        body,
        out_type=jax.ShapeDtypeStruct((E, D), jnp.float32),
        mesh=_mesh(),
        scratch_types=(
            [pltpu.VMEM((epw,), jnp.int32)]
            + [pltpu.VMEM((GBG, D), jnp.float32) for _ in range(4)]
            + [
                pltpu.VMEM((max(tail, 8),), jnp.int32),
                pltpu.VMEM((max(tail, 8), D), jnp.float32),
                pltpu.VMEM_SHARED((N, D), jnp.float32),
            ]
            + [pltpu.SemaphoreType.DMA for _ in range(9)]
        ),
    )


@functools.lru_cache(maxsize=None)
def _make_segsum(N, E, D):
    epw = E // NW
    nb = epw // GB
    tail = epw - nb * GB
    rpt = -(-(N // NS) // 8) * 8
    rlast = N - (NS - 1) * rpt
    assert E % NW == 0 and tail % 8 == 0 and 0 < rlast <= rpt

    GBS = 64
    nb = epw // GBS
    tail = epw - nb * GBS
    assert tail % 8 == 0
    R = 4
    DIST = 2
    T = nb // R
    nb8 = T * R
    ZR = 56

    def body(t_hbm, dst_hbm, agg_hbm, b0, b1, b2, b3, i0, i1, i2, i3,
             buf_t, idx_t, zbuf, acc,
             rs0, rs1, rs2, rs3, ss0, ss1, ss2, ss3, sem):
        c = lax.axis_index("c")
        s = lax.axis_index("s")
        bufs = (b0, b1, b2, b3)
        ibufs = (i0, i1, i2, i3)
        rsems = (rs0, rs1, rs2, rs3)
        ssems = (ss0, ss1, ss2, ss3)

        def zstep(i, carry):
            r = i // (D // 16)
            q = (i % (D // 16)) * 16
            zbuf[r, pl.ds(q, 16)] = jnp.zeros((16,), jnp.float32)
            return carry

        lax.fori_loop(0, ZR * (D // 16), zstep, 0)
        r0 = pl.multiple_of(s * rpt, 8)
        nrep = lax.select(s < NS - 1, rpt // ZR, rlast // ZR)

        def zcopy(r, carry):
            pltpu.sync_copy(zbuf, acc.at[pl.ds(r0 + r * ZR, ZR)])
            return carry

        lax.fori_loop(0, nrep, zcopy, 0)
        zr_done = nrep * ZR
        zrem1 = rpt - (rpt // ZR) * ZR
        zrem2 = rlast - (rlast // ZR) * ZR
        assert zrem1 == zrem2 and zrem1 > 0
        pltpu.sync_copy(zbuf.at[pl.ds(0, zrem1)], acc.at[pl.ds(r0 + zr_done, zrem1)])
        plsc.subcore_barrier()

        base = pl.multiple_of(((c * NS + s) * epw).astype(jnp.int32), 8)

        def fire_reads(j, b):
            off = pl.multiple_of(base + j * GBS, 8)
            pltpu.async_copy(t_hbm.at[pl.ds(off, GBS)], bufs[b], rsems[b])
            pltpu.async_copy(dst_hbm.at[pl.ds(off, GBS)], ibufs[b], rsems[b])

        def wait_reads(b):
            pltpu.make_async_copy(t_hbm.at[pl.ds(0, GBS)], bufs[b], rsems[b]).wait()
            pltpu.make_async_copy(dst_hbm.at[pl.ds(0, GBS)], ibufs[b], rsems[b]).wait()

        def fire_scat(b):
            pltpu.async_copy(bufs[b], acc.at[ibufs[b]], ssems[b], add=True)

        def wait_scat(b):
            pltpu.make_async_copy(t_hbm.at[pl.ds(0, GBS)], bufs[b], ssems[b]).wait()

        if T:
            for b in range(DIST):
                fire_reads(b, b)

            def outer(t, carry):
                j0 = t * R
                for b in range(R):
                    j = j0 + b
                    jw = j - DIST
                    bw = (b - DIST) % R

                    @pl.when(j - R >= 0)
                    def _():
                        wait_scat(b)

                    @pl.when(j >= DIST)
                    def _():
                        fire_reads(j, b)

                    @pl.when(jw >= 0)
                    def _():
                        wait_reads(bw)
                        fire_scat(bw)
                return carry

            lax.fori_loop(0, T, outer, 0)
            for j in range(nb8 - DIST, nb8):
                b = j % R
                wait_reads(b)
                fire_scat(b)
            for j in range(max(nb8 - R, 0), nb8):
                wait_scat(j % R)
        for j in range(nb8, nb):
            off = pl.multiple_of(base + j * GBS, 8)
            pltpu.sync_copy(t_hbm.at[pl.ds(off, GBS)], b0)
            pltpu.sync_copy(dst_hbm.at[pl.ds(off, GBS)], i0)
            pltpu.sync_copy(b0, acc.at[i0], add=True)
        if tail:
            off = pl.multiple_of(base + nb * GBS, 8)
            pltpu.sync_copy(t_hbm.at[pl.ds(off, tail)], buf_t)
            pltpu.sync_copy(dst_hbm.at[pl.ds(off, tail)], idx_t)
            pltpu.sync_copy(buf_t, acc.at[idx_t], add=True)
        plsc.subcore_barrier()

        @pl.when(s < NS - 1)
        def _():
            pltpu.sync_copy(
                acc.at[pl.ds(r0, rpt)], agg_hbm.at[c, pl.ds(r0, rpt)]
            )

        @pl.when(s == NS - 1)
        def _():
            pltpu.sync_copy(
                acc.at[pl.ds(r0, rlast)], agg_hbm.at[c, pl.ds(r0, rlast)]
            )

    return pl.kernel(
        body,
        out_type=jax.ShapeDtypeStruct((NC, N, D), jnp.float32),
        mesh=_mesh(),
        scratch_types=(
            [pltpu.VMEM((GBS, D), jnp.float32) for _ in range(4)]
            + [pltpu.VMEM((GBS,), jnp.int32) for _ in range(4)]
            + [
                pltpu.VMEM((max(tail, 8), D), jnp.float32),
                pltpu.VMEM((max(tail, 8),), jnp.int32),
                pltpu.VMEM((ZR, D), jnp.float32),
                pltpu.VMEM_SHARED((N, D), jnp.float32),
            ]
            + [pltpu.SemaphoreType.DMA for _ in range(9)]
        ),
    )


def _edge_transform(xj, bond, wstack, off_rows=0):
    D = xj.shape[1]
    rows = bond.shape[0]
    BDp1 = wstack.shape[0]
    be = 2048
    assert off_rows % be == 0
    ob = off_rows // be
    grid = -(-rows // be)

    def body(xj_ref, b_ref, w_ref, o_ref):
        x = xj_ref[...]
        acc = jnp.dot(x, w_ref[0], preferred_element_type=jnp.float32)
        for k in range(1, BDp1):
            acc += b_ref[:, k - 1 : k] * jnp.dot(
                x, w_ref[k], preferred_element_type=jnp.float32
            )
        o_ref[...] = acc

    return pl.pallas_call(
        body,
        grid=(grid,),
        in_specs=[
            pl.BlockSpec((be, D), lambda i: (i + ob, 0)),
            pl.BlockSpec((be, bond.shape[1]), lambda i: (i, 0)),
            pl.BlockSpec((BDp1, D, D), lambda i: (0, 0, 0)),
        ],
        out_specs=pl.BlockSpec((be, D), lambda i: (i, 0)),
        out_shape=jax.ShapeDtypeStruct((rows, D), jnp.float32),
    )(xj, bond, wstack)


def _gru(aggs, h, wk, wr, b):
    N, D = h.shape
    bn = 2000
    assert N % bn == 0
    na = len(aggs)

    def body(*refs):
        a_refs = refs[:na]
        h_ref, wk_ref, wr_ref, b_ref, o_ref = refs[na:]
        a = a_refs[0][0] + a_refs[0][1]
        for ar in a_refs[1:]:
            a = a + ar[0] + ar[1]
        hh = h_ref[...]
        xp = jnp.dot(a, wk_ref[...], preferred_element_type=jnp.float32) + b_ref[0]
        hp = jnp.dot(hh, wr_ref[...], preferred_element_type=jnp.float32) + b_ref[1]
        z = jax.nn.sigmoid(xp[:, :D] + hp[:, :D])
        r = jax.nn.sigmoid(xp[:, D : 2 * D] + hp[:, D : 2 * D])
        cand = jnp.tanh(xp[:, 2 * D :] + r * hp[:, 2 * D :])
        o_ref[...] = z * hh + (1.0 - z) * cand

    return pl.pallas_call(
        body,
        grid=(N // bn,),
        in_specs=[pl.BlockSpec((NC, bn, D), lambda i: (0, i, 0)) for _ in range(na)]
        + [
            pl.BlockSpec((bn, D), lambda i: (i, 0)),
            pl.BlockSpec((D, 3 * D), lambda i: (0, 0)),
            pl.BlockSpec((D, 3 * D), lambda i: (0, 0)),
            pl.BlockSpec((2, 3 * D), lambda i: (0, 0)),
        ],
        out_specs=pl.BlockSpec((bn, D), lambda i: (i, 0)),
        out_shape=jax.ShapeDtypeStruct((N, D), jnp.float32),
    )(*aggs, h, wk, wr, b)


def kernel(atom_features, bond_features, pair_indices, kernel, bias, gru_kernel,
           gru_rec_kernel, gru_bias):
    N, D = atom_features.shape
    E, BD = bond_features.shape
    dst = pair_indices[:, 0].astype(jnp.int32)
    src = pair_indices[:, 1].astype(jnp.int32)
    Kr = kernel.reshape(BD, D, D)
    wstack = jnp.concatenate(
        [bias.reshape(D, D).T[None], jnp.transpose(Kr, (0, 2, 1))], axis=0
    )
    EA = (E // (2 * NW * GB)) * (NW * GB)
    if 0 < EA < E:
        bounds = [(0, EA), (EA, E)]
    else:
        bounds = [(0, E)]
    parts = [
        (src[lo:hi], dst[lo:hi], bond_features[lo:hi],
         _make_gather(N, hi - lo, D), _make_segsum(N, hi - lo, D))
        for lo, hi in bounds
    ]

    h = atom_features
    for _ in range(_STEPS):
        xjs = [g(h, s_) for s_, _, _, g, _ in parts]
        aggs = []
        for i, (s_, d_, bf, g, seg) in enumerate(parts):
            t = _edge_transform(xjs[i], bf, wstack)
            aggs.append(seg(t, d_))
        h = _gru(aggs, h, gru_kernel, gru_rec_kernel, gru_bias)
    return h
